# baseline (speedup 1.0000x reference)
import functools

import jax
import jax.numpy as jnp
from jax import lax
from jax.experimental import pallas as pl
from jax.experimental.pallas import tpu as pltpu
from jax.experimental.pallas import tpu_sc as plsc

B, C, H, W = 4, 192, 224, 224
PLANES = B * C
NC, NS = 2, 16
NW = NC * NS
NG = PLANES // NW

_MESH = plsc.VectorSubcoreMesh(core_axis_name="c", subcore_axis_name="s")


@functools.partial(
    pl.kernel,
    out_type=jax.ShapeDtypeStruct((PLANES, H, W), jnp.float32),
    mesh=_MESH,
    scratch_types=[
        pltpu.VMEM((32,), jnp.int32),
        pltpu.SemaphoreType.DMA,
    ],
)
def _permute_planes(in_hbm, idx_hbm, out_hbm, idx_v, sem):
    wid = lax.axis_index("s") * NC + lax.axis_index("c")
    base = wid * NG
    pltpu.sync_copy(idx_hbm.at[wid], idx_v)

    lo, hi = idx_v[pl.ds(0, 16)], idx_v[pl.ds(16, 16)]

    def src(g):
        return lo[g] if g < 16 else hi[g - 16]

    for g in range(NG):
        pltpu.async_copy(
            in_hbm.at[pl.ds(src(g), 1)], out_hbm.at[pl.ds(base + g, 1)], sem)
    for g in range(NG):
        pltpu.make_async_copy(
            in_hbm.at[pl.ds(0, 1)], out_hbm.at[pl.ds(base + g, 1)], sem).wait()


def kernel(input, indices):
    src_plane = (jnp.arange(B, dtype=jnp.int32)[:, None] * C
                 + indices[None, :].astype(jnp.int32))
    idx = jnp.pad(src_plane.reshape(NW, NG), ((0, 0), (0, 32 - NG)))
    out = _permute_planes(input.reshape(PLANES, H, W), idx)
    return out.reshape(input.shape), 0.0


# speedup vs baseline: 36.3485x; 36.3485x over previous
import functools

import jax
import jax.numpy as jnp
from jax import lax
from jax.experimental import pallas as pl
from jax.experimental.pallas import tpu as pltpu
from jax.experimental.pallas import tpu_sc as plsc

B, C, H, W = 4, 192, 224, 224
PLANES = B * C
NC, NS = 2, 16
NW = NC * NS
NG = PLANES // NW
HH = H // 2
NCHUNK = NG * 2
NBUF = 4

_MESH = plsc.VectorSubcoreMesh(core_axis_name="c", subcore_axis_name="s")


@functools.partial(
    pl.kernel,
    out_type=jax.ShapeDtypeStruct((PLANES, H, W), jnp.float32),
    mesh=_MESH,
    scratch_types=(
        [pltpu.VMEM((32,), jnp.int32)]
        + [pltpu.VMEM((1, HH, W), jnp.float32) for _ in range(NBUF)]
        + [pltpu.SemaphoreType.DMA for _ in range(2 * NBUF)]
    ),
)
def _permute_planes(in_hbm, idx_hbm, out_hbm, idx_v, *rest):
    bufs = rest[:NBUF]
    gsem = rest[NBUF:2 * NBUF]
    ssem = rest[2 * NBUF:]
    wid = lax.axis_index("s") * NC + lax.axis_index("c")
    base = wid * NG
    pltpu.sync_copy(idx_hbm.at[wid], idx_v)

    lo, hi = idx_v[pl.ds(0, 16)], idx_v[pl.ds(16, 16)]

    def src_slice(k):
        g, half = k // 2, k % 2
        p = lo[g] if g < 16 else hi[g - 16]
        return in_hbm.at[pl.ds(p, 1), pl.ds(half * HH, HH)]

    def dst_slice(k):
        g, half = k // 2, k % 2
        return out_hbm.at[pl.ds(base + g, 1), pl.ds(half * HH, HH)]

    for b in range(NBUF):
        pltpu.async_copy(src_slice(b), bufs[b], gsem[b])

    for k in range(NCHUNK):
        b = k % NBUF
        pltpu.make_async_copy(in_hbm.at[pl.ds(0, 1), pl.ds(0, HH)],
                              bufs[b], gsem[b]).wait()
        pltpu.async_copy(bufs[b], dst_slice(k), ssem[b])
        if k + NBUF < NCHUNK:
            pltpu.make_async_copy(bufs[b], dst_slice(k), ssem[b]).wait()
            pltpu.async_copy(src_slice(k + NBUF), bufs[b], gsem[b])

    for k in range(NCHUNK - NBUF, NCHUNK):
        b = k % NBUF
        pltpu.make_async_copy(bufs[b], dst_slice(k), ssem[b]).wait()


def kernel(input, indices):
    src_plane = (jnp.arange(B, dtype=jnp.int32)[:, None] * C
                 + indices[None, :].astype(jnp.int32))
    idx = jnp.pad(src_plane.reshape(NW, NG), ((0, 0), (0, 32 - NG)))
    out = _permute_planes(input.reshape(PLANES, H, W), idx)
    return out.reshape(input.shape), 0.0


# speedup vs baseline: 39.4581x; 1.0855x over previous
import functools

import jax
import jax.numpy as jnp
from jax import lax
from jax.experimental import pallas as pl
from jax.experimental.pallas import tpu as pltpu
from jax.experimental.pallas import tpu_sc as plsc

B, C, H, W = 4, 192, 224, 224
PLANES = B * C
NC, NS = 2, 16
NW = NC * NS
NG = PLANES // NW

_MESH = plsc.VectorSubcoreMesh(core_axis_name="c", subcore_axis_name="s")


@functools.partial(
    pl.kernel,
    out_type=jax.ShapeDtypeStruct((PLANES, H, W), jnp.float32),
    mesh=_MESH,
    scratch_types=[
        pltpu.VMEM((32,), jnp.int32),
        pltpu.VMEM_SHARED((NS, 2, 1, H, W), jnp.float32),
        pltpu.SemaphoreType.DMA,
        pltpu.SemaphoreType.DMA,
        pltpu.SemaphoreType.DMA,
        pltpu.SemaphoreType.DMA,
    ],
)
def _permute_planes(in_hbm, idx_hbm, out_hbm, idx_v, shared, g0, g1, s0, s1):
    wid = lax.axis_index("s") * NC + lax.axis_index("c")
    sid = lax.axis_index("s")
    base = wid * NG
    pltpu.sync_copy(idx_hbm.at[wid], idx_v)
    bufs = (shared.at[sid, 0], shared.at[sid, 1])
    gsem = (g0, g1)
    ssem = (s0, s1)

    lo, hi = idx_v[pl.ds(0, 16)], idx_v[pl.ds(16, 16)]

    def src(g):
        return lo[g] if g < 16 else hi[g - 16]

    for b in range(2):
        pltpu.async_copy(in_hbm.at[pl.ds(src(b), 1)], bufs[b], gsem[b])

    for g in range(NG):
        b = g & 1
        pltpu.make_async_copy(
            in_hbm.at[pl.ds(0, 1)], bufs[b], gsem[b]).wait()
        pltpu.async_copy(bufs[b], out_hbm.at[pl.ds(base + g, 1)], ssem[b])
        if g + 2 < NG:
            pltpu.make_async_copy(
                bufs[b], out_hbm.at[pl.ds(base + g, 1)], ssem[b]).wait()
            pltpu.async_copy(
                in_hbm.at[pl.ds(src(g + 2), 1)], bufs[b], gsem[b])

    for g in (NG - 2, NG - 1):
        b = g & 1
        pltpu.make_async_copy(
            bufs[b], out_hbm.at[pl.ds(base + g, 1)], ssem[b]).wait()


def kernel(input, indices):
    src_plane = (jnp.arange(B, dtype=jnp.int32)[:, None] * C
                 + indices[None, :].astype(jnp.int32))
    idx = jnp.pad(src_plane.reshape(NW, NG), ((0, 0), (0, 32 - NG)))
    out = _permute_planes(input.reshape(PLANES, H, W), idx)
    return out.reshape(input.shape), 0.0
